# Initial kernel scaffold; baseline (speedup 1.0000x reference)
import functools

import jax
import jax.numpy as jnp
from jax import lax
from jax.experimental import pallas as pl
from jax.experimental.pallas import tpu as pltpu
from jax.experimental.pallas import tpu_sc as plsc

_N_ENT = 10000
_D = 128
_N_RELSLOT = 41
_REL_PAD = 48
_B = 4
_NW = 32
_CHUNK = 128
_N_CHUNKS = 40
_E_PER_W = _CHUNK * _N_CHUNKS
_E_PAD = _NW * _E_PER_W
_ACC_ROWS = 10240
_ROWS_PER_TILE = _ACC_ROWS // 16


def _rel_body(q_ref, w_ref, b_ref, out_ref):
    r = lax.dot_general(
        q_ref[...], w_ref[...], (((1,), (1,)), ((), ())),
        preferred_element_type=jnp.float32,
    )
    r = r + b_ref[...]
    out_ref[...] = r.reshape(_B, _REL_PAD, _D)


def _relation_table(query, rel_W_pad, rel_b_pad):
    return pl.pallas_call(
        _rel_body,
        out_shape=jax.ShapeDtypeStruct((_B, _REL_PAD, _D), jnp.float32),
    )(query, rel_W_pad, rel_b_pad)


def _sc_body(li, rel_tab, subr, relr, objr, out,
             sub_v, rel_v, obj_v, in_v, rw_v, msg_v, zb_v, acc, sem_a, sem_b):
    c_id = lax.axis_index("c")
    s_id = lax.axis_index("s")
    wid = s_id * 2 + c_id
    pltpu.sync_copy(subr.at[pl.ds(wid * _N_CHUNKS, _N_CHUNKS)], sub_v)
    pltpu.sync_copy(relr.at[pl.ds(wid * _N_CHUNKS, _N_CHUNKS)], rel_v)
    pltpu.sync_copy(objr.at[pl.ds(wid * _N_CHUNKS, _N_CHUNKS)], obj_v)

    def _zrow(i, carry):
        for j in range(8):
            zb_v[i, pl.ds(j * 16, 16)] = jnp.zeros((16,), jnp.float32)
        return carry

    lax.fori_loop(0, _CHUNK, _zrow, 0)

    row0 = s_id * _ROWS_PER_TILE
    for b in range(_B):
        for k in range(5):
            pltpu.sync_copy(zb_v, acc.at[pl.ds(row0 + k * _CHUNK, _CHUNK)])
        plsc.subcore_barrier()

        def _chunk(c, carry):
            ga = pltpu.async_copy(li.at[b].at[sub_v.at[c]], in_v, sem_a)
            gb = pltpu.async_copy(rel_tab.at[b].at[rel_v.at[c]], rw_v, sem_b)
            ga.wait()
            gb.wait()

            def _mrow(i, cc):
                for j in range(8):
                    msg_v[i, pl.ds(j * 16, 16)] = (
                        in_v[i, pl.ds(j * 16, 16)] * rw_v[i, pl.ds(j * 16, 16)]
                    )
                return cc

            lax.fori_loop(0, _CHUNK, _mrow, 0)
            pltpu.sync_copy(msg_v, acc.at[obj_v.at[c]], add=True)
            return carry

        lax.fori_loop(0, _N_CHUNKS, _chunk, 0)
        plsc.subcore_barrier()
        for k in range(5):
            pltpu.sync_copy(
                acc.at[pl.ds(row0 + k * _CHUNK, _CHUNK)],
                out.at[c_id, b, pl.ds(row0 + k * _CHUNK, _CHUNK)],
            )
        plsc.subcore_barrier()


def _sc_aggregate(li, rel_tab, sub2d, rel2d, obj2d):
    mesh = plsc.VectorSubcoreMesh(core_axis_name="c", subcore_axis_name="s")
    fn = functools.partial(
        pl.kernel,
        out_type=jax.ShapeDtypeStruct((2, _B, _ACC_ROWS, _D), jnp.float32),
        mesh=mesh,
        scratch_types=[
            pltpu.VMEM((_N_CHUNKS, _CHUNK), jnp.int32),
            pltpu.VMEM((_N_CHUNKS, _CHUNK), jnp.int32),
            pltpu.VMEM((_N_CHUNKS, _CHUNK), jnp.int32),
            pltpu.VMEM((_CHUNK, _D), jnp.float32),
            pltpu.VMEM((_CHUNK, _D), jnp.float32),
            pltpu.VMEM((_CHUNK, _D), jnp.float32),
            pltpu.VMEM((_CHUNK, _D), jnp.float32),
            pltpu.VMEM_SHARED((_ACC_ROWS, _D), jnp.float32),
            pltpu.SemaphoreType.DMA,
            pltpu.SemaphoreType.DMA,
        ],
    )(_sc_body)
    return fn(li, rel_tab, sub2d, rel2d, obj2d)


def _fin_body(p0_ref, p1_ref, bd_ref, li_ref, w1_ref, w2_ref, b_ref, out_ref):
    x = p0_ref[0] + p1_ref[0] + bd_ref[0]
    li = li_ref[0]
    y = (
        lax.dot_general(x, w1_ref[...], (((1,), (1,)), ((), ())),
                        preferred_element_type=jnp.float32)
        + lax.dot_general(li, w2_ref[...], (((1,), (1,)), ((), ())),
                          preferred_element_type=jnp.float32)
        + b_ref[...]
    )
    mu = jnp.mean(y, axis=-1, keepdims=True)
    yc = y - mu
    var = jnp.mean(yc * yc, axis=-1, keepdims=True)
    z = yc * lax.rsqrt(var + 1e-5)
    out_ref[0] = jnp.maximum(z, 0.0)


def _finalize(p0, p1, boundary, layer_input, w1, w2, b2d):
    br = 1000
    grid = (_B, _N_ENT // br)
    blk = lambda: pl.BlockSpec((1, br, _D), lambda b, i: (b, i, 0))
    wblk = pl.BlockSpec((_D, _D), lambda b, i: (0, 0))
    return pl.pallas_call(
        _fin_body,
        grid=grid,
        in_specs=[
            blk(), blk(), blk(), blk(),
            wblk, wblk,
            pl.BlockSpec((1, _D), lambda b, i: (0, 0)),
        ],
        out_specs=blk(),
        out_shape=jax.ShapeDtypeStruct((_B, _N_ENT, _D), jnp.float32),
    )(p0, p1, boundary, layer_input, w1, w2, b2d)


def kernel(query, layer_input, edges, n_ent, boundary, rel_W, rel_b, Wh_W, Wh_b):
    edges = edges.astype(jnp.int32)
    sub = edges[:, 0]
    rel = edges[:, 1]
    obj = edges[:, 2]
    npad = _E_PAD - sub.shape[0]
    sub2d = jnp.concatenate([sub, jnp.zeros((npad,), jnp.int32)]).reshape(-1, _CHUNK)
    rel2d = jnp.concatenate(
        [rel, jnp.full((npad,), _N_RELSLOT, jnp.int32)]).reshape(-1, _CHUNK)
    obj2d = jnp.concatenate([obj, jnp.zeros((npad,), jnp.int32)]).reshape(-1, _CHUNK)

    w3 = rel_W.reshape(_N_RELSLOT, _D, _D)
    w3 = jnp.pad(w3, ((0, _REL_PAD - _N_RELSLOT), (0, 0), (0, 0)))
    rel_W_pad = w3.reshape(_REL_PAD * _D, _D)
    rel_b_pad = jnp.pad(rel_b, (0, (_REL_PAD - _N_RELSLOT) * _D)).reshape(1, -1)

    rel_tab = _relation_table(query, rel_W_pad, rel_b_pad)
    partials = _sc_aggregate(layer_input, rel_tab, sub2d, rel2d, obj2d)
    p0 = partials[0, :, :_N_ENT, :]
    p1 = partials[1, :, :_N_ENT, :]

    w1 = Wh_W[:, :_D]
    w2 = Wh_W[:, _D:]
    return _finalize(p0, p1, boundary, layer_input, w1, w2, Wh_b.reshape(1, _D))


# speedup vs baseline: 14.6656x; 14.6656x over previous
import functools

import jax
import jax.numpy as jnp
from jax import lax
from jax.experimental import pallas as pl
from jax.experimental.pallas import tpu as pltpu
from jax.experimental.pallas import tpu_sc as plsc

_N_ENT = 10000
_D = 128
_N_RELSLOT = 41
_REL_PAD = 48
_B = 4
_NW = 32
_CHUNK = 128
_N_CHUNKS = 40
_E_PER_W = _CHUNK * _N_CHUNKS
_E_PAD = _NW * _E_PER_W
_ACC_ROWS = 10240
_ROWS_PER_TILE = _ACC_ROWS // 16


def _rel_body(q_ref, w_ref, b_ref, out_ref):
    r = lax.dot_general(
        q_ref[...], w_ref[...], (((1,), (1,)), ((), ())),
        preferred_element_type=jnp.float32,
    )
    r = r + b_ref[...]
    out_ref[...] = r.reshape(_B, _REL_PAD, _D)


def _relation_table(query, rel_W_pad, rel_b_pad):
    return pl.pallas_call(
        _rel_body,
        out_shape=jax.ShapeDtypeStruct((_B, _REL_PAD, _D), jnp.float32),
    )(query, rel_W_pad, rel_b_pad)


def _sc_body(li, rel_tab, subr, relr, objr, out,
             sub_v, rel_v, obj_v, in_v, rw_v, acc, sem_a, sem_b):
    c_id = lax.axis_index("c")
    s_id = lax.axis_index("s")
    wid = s_id * 2 + c_id
    pltpu.sync_copy(subr.at[pl.ds(wid * _N_CHUNKS, _N_CHUNKS)], sub_v)
    pltpu.sync_copy(relr.at[pl.ds(wid * _N_CHUNKS, _N_CHUNKS)], rel_v)
    pltpu.sync_copy(objr.at[pl.ds(wid * _N_CHUNKS, _N_CHUNKS)], obj_v)

    row0 = s_id * _ROWS_PER_TILE
    for b in range(_B):
        def _zrow(i, carry):
            for j in range(8):
                rw_v[i, pl.ds(j * 16, 16)] = jnp.zeros((16,), jnp.float32)
            return carry

        lax.fori_loop(0, _CHUNK, _zrow, 0)
        for k in range(5):
            pltpu.sync_copy(rw_v, acc.at[pl.ds(row0 + k * _CHUNK, _CHUNK)])
        plsc.subcore_barrier()

        def _chunk(c, carry):
            ga = pltpu.async_copy(li.at[b].at[sub_v.at[c]], in_v, sem_a)
            gb = pltpu.async_copy(rel_tab.at[b].at[rel_v.at[c]], rw_v, sem_b)
            ga.wait()
            gb.wait()

            def _mrow(i, cc):
                for j in range(8):
                    in_v[i, pl.ds(j * 16, 16)] = (
                        in_v[i, pl.ds(j * 16, 16)] * rw_v[i, pl.ds(j * 16, 16)]
                    )
                return cc

            lax.fori_loop(0, _CHUNK, _mrow, 0)
            pltpu.sync_copy(in_v, acc.at[obj_v.at[c]], add=True)
            return carry

        lax.fori_loop(0, _N_CHUNKS, _chunk, 0)
        plsc.subcore_barrier()
        for k in range(5):
            pltpu.sync_copy(
                acc.at[pl.ds(row0 + k * _CHUNK, _CHUNK)],
                out.at[c_id, b, pl.ds(row0 + k * _CHUNK, _CHUNK)],
            )
        plsc.subcore_barrier()


def _sc_aggregate(li, rel_tab, sub2d, rel2d, obj2d):
    mesh = plsc.VectorSubcoreMesh(core_axis_name="c", subcore_axis_name="s")
    fn = functools.partial(
        pl.kernel,
        out_type=jax.ShapeDtypeStruct((2, _B, _ACC_ROWS, _D), jnp.float32),
        mesh=mesh,
        scratch_types=[
            pltpu.VMEM((_N_CHUNKS, _CHUNK), jnp.int32),
            pltpu.VMEM((_N_CHUNKS, _CHUNK), jnp.int32),
            pltpu.VMEM((_N_CHUNKS, _CHUNK), jnp.int32),
            pltpu.VMEM((_CHUNK, _D), jnp.float32),
            pltpu.VMEM((_CHUNK, _D), jnp.float32),
            pltpu.VMEM_SHARED((_ACC_ROWS, _D), jnp.float32),
            pltpu.SemaphoreType.DMA,
            pltpu.SemaphoreType.DMA,
        ],
    )(_sc_body)
    return fn(li, rel_tab, sub2d, rel2d, obj2d)


def _fin_body(p0_ref, p1_ref, bd_ref, li_ref, w1_ref, w2_ref, b_ref, out_ref):
    x = p0_ref[0] + p1_ref[0] + bd_ref[0]
    li = li_ref[0]
    y = (
        lax.dot_general(x, w1_ref[...], (((1,), (1,)), ((), ())),
                        preferred_element_type=jnp.float32)
        + lax.dot_general(li, w2_ref[...], (((1,), (1,)), ((), ())),
                          preferred_element_type=jnp.float32)
        + b_ref[...]
    )
    mu = jnp.mean(y, axis=-1, keepdims=True)
    yc = y - mu
    var = jnp.mean(yc * yc, axis=-1, keepdims=True)
    z = yc * lax.rsqrt(var + 1e-5)
    out_ref[0] = jnp.maximum(z, 0.0)


def _finalize(p0, p1, boundary, layer_input, w1, w2, b2d):
    br = 1000
    grid = (_B, _N_ENT // br)
    blk = lambda: pl.BlockSpec((1, br, _D), lambda b, i: (b, i, 0))
    wblk = pl.BlockSpec((_D, _D), lambda b, i: (0, 0))
    return pl.pallas_call(
        _fin_body,
        grid=grid,
        in_specs=[
            blk(), blk(), blk(), blk(),
            wblk, wblk,
            pl.BlockSpec((1, _D), lambda b, i: (0, 0)),
        ],
        out_specs=blk(),
        out_shape=jax.ShapeDtypeStruct((_B, _N_ENT, _D), jnp.float32),
    )(p0, p1, boundary, layer_input, w1, w2, b2d)


def kernel(query, layer_input, edges, n_ent, boundary, rel_W, rel_b, Wh_W, Wh_b):
    edges = edges.astype(jnp.int32)
    sub = edges[:, 0]
    rel = edges[:, 1]
    obj = edges[:, 2]
    npad = _E_PAD - sub.shape[0]
    sub2d = jnp.concatenate([sub, jnp.zeros((npad,), jnp.int32)]).reshape(-1, _CHUNK)
    rel2d = jnp.concatenate(
        [rel, jnp.full((npad,), _N_RELSLOT, jnp.int32)]).reshape(-1, _CHUNK)
    obj2d = jnp.concatenate([obj, jnp.zeros((npad,), jnp.int32)]).reshape(-1, _CHUNK)

    w3 = rel_W.reshape(_N_RELSLOT, _D, _D)
    w3 = jnp.pad(w3, ((0, _REL_PAD - _N_RELSLOT), (0, 0), (0, 0)))
    rel_W_pad = w3.reshape(_REL_PAD * _D, _D)
    rel_b_pad = jnp.pad(rel_b, (0, (_REL_PAD - _N_RELSLOT) * _D)).reshape(1, -1)

    rel_tab = _relation_table(query, rel_W_pad, rel_b_pad)
    partials = _sc_aggregate(layer_input, rel_tab, sub2d, rel2d, obj2d)
    p0 = partials[0, :, :_N_ENT, :]
    p1 = partials[1, :, :_N_ENT, :]

    w1 = Wh_W[:, :_D]
    w2 = Wh_W[:, _D:]
    return _finalize(p0, p1, boundary, layer_input, w1, w2, Wh_b.reshape(1, _D))
